# baseline (speedup 1.0000x reference)
import functools

import jax
import jax.numpy as jnp
from jax import lax
from jax.experimental import pallas as pl
from jax.experimental.pallas import tpu as pltpu
from jax.experimental.pallas import tpu_sc as plsc

DIM = 64
LUT_SIZE = 100000
CQ_SIZE = 5000
MOMENTUM = 0.5
SCALAR = 30.0
MARGIN = 0.25
BATCH = 1024

_T = 4000
_NLUT = LUT_SIZE // _T
_BIG = 1e30
_LOG2E = 1.4426950408889634


def _sc_gather(lut, safe_lab):
  mesh = plsc.VectorSubcoreMesh(core_axis_name="c", subcore_axis_name="s")
  n_per = BATCH // 32

  @functools.partial(
      pl.kernel,
      out_type=jax.ShapeDtypeStruct((BATCH, DIM), jnp.float32),
      mesh=mesh,
      scratch_types=[
          pltpu.VMEM((n_per,), jnp.int32),
          pltpu.VMEM((n_per, DIM), jnp.float32),
          pltpu.SemaphoreType.DMA,
      ],
  )
  def gather_k(table_hbm, idx_hbm, out_hbm, idx_v, rows_v, sem):
    wid = lax.axis_index("s") * 2 + lax.axis_index("c")
    base = wid * n_per
    pltpu.sync_copy(idx_hbm.at[pl.ds(base, n_per)], idx_v)
    handles = []
    for c in range(n_per // 16):
      vec = idx_v[pl.ds(c * 16, 16)]
      for k in range(16):
        j = c * 16 + k
        handles.append(pltpu.async_copy(
            table_hbm.at[pl.ds(vec[k], 1), :], rows_v.at[pl.ds(j, 1), :],
            sem))
    for h in handles:
      h.wait()
    pltpu.sync_copy(rows_v, out_hbm.at[pl.ds(base, n_per)])

  return gather_k(lut, safe_lab)


def _prep_step(vec_ref, labc_ref, labr_ref, g_ref, cq_ref, newcq_ref,
               nv30_s, delta_s, winlab_s, lmask_s, scal_s, se_s, buf_s):
  f32 = jnp.float32
  v = vec_ref[...]
  nrm = jnp.sqrt(jnp.sum(v * v, axis=1, keepdims=True))
  nv = v / jnp.maximum(nrm, 1e-12)
  g = g_ref[...]
  nv30 = (nv * (SCALAR * _LOG2E)).astype(jnp.bfloat16)
  nv30_s[...] = nv30

  secq = jnp.zeros((BATCH, 1), f32)
  for c in range(CQ_SIZE // 1000):
    chunk = cq_ref[pl.ds(c * 1000, 1000), :].astype(jnp.bfloat16)
    s_c = lax.dot_general(nv30, chunk, (((1,), (1,)), ((), ())),
                          preferred_element_type=f32)
    secq = secq + jnp.sum(jnp.exp2(s_c), axis=1, keepdims=True)
  se_s[...] = secq

  labc = labc_ref[...]
  labr = labr_ref[...]
  lab_c = labc - 1
  lab_r = labr - 1
  l_c = (labc > 0) & (lab_c < LUT_SIZE)
  l_r = (labr > 0) & (lab_r < LUT_SIZE)
  u_c = (labc > 0) & (lab_c >= LUT_SIZE)
  u_r = (labr > 0) & (lab_r >= LUT_SIZE)

  eq = (lab_c == lab_r) & l_c & l_r
  io_c = lax.broadcasted_iota(jnp.int32, (BATCH, BATCH), 0)
  io_r = lax.broadcasted_iota(jnp.int32, (BATCH, BATCH), 1)
  rep_r = l_r & ~jnp.any(eq & (io_c < io_r), axis=0, keepdims=True)
  rep_c = l_c & ~jnp.any(eq & (io_r < io_c), axis=1, keepdims=True)
  win_r = l_r & ~jnp.any(eq & (io_c > io_r), axis=0, keepdims=True)
  winlab_s[...] = jnp.where(win_r, lab_r, -1)

  ratio = (1.0 - MOMENTUM) / MOMENTUM
  w = g + ratio * nv
  wn = jnp.sqrt(jnp.sum(w * w, axis=1, keepdims=True))
  upd = w / jnp.maximum(wn, 1e-12)
  delta_s[...] = (upd - g).astype(jnp.bfloat16)

  dot_t = lambda a, b: lax.dot_general(a, b, (((1,), (1,)), ((), ())),
                                       preferred_element_type=f32)
  s_nn = dot_t(nv, nv)
  s_ng = dot_t(nv, g)
  s_gn = dot_t(g, nv)
  s_gg = dot_t(g, g)

  neq = ~eq
  m_negb = (l_r & neq) | u_r
  m_posc = eq & rep_r
  m_negc = rep_r & neq

  pos_b = jnp.minimum(
      jnp.min(jnp.where(eq, s_nn, _BIG), axis=1, keepdims=True),
      jnp.min(jnp.where(m_posc, s_ng, _BIG), axis=1, keepdims=True))
  neg_b = jnp.maximum(
      jnp.max(jnp.where(m_negb, s_nn, -_BIG), axis=1, keepdims=True),
      jnp.max(jnp.where(m_negc, s_ng, -_BIG), axis=1, keepdims=True))
  term_b = jnp.where(l_c, jax.nn.relu(neg_b - pos_b + MARGIN), 0.0)

  pos_c = jnp.minimum(
      jnp.min(jnp.where(eq, s_gn, _BIG), axis=1, keepdims=True),
      jnp.min(jnp.where(m_posc, s_gg, _BIG), axis=1, keepdims=True))
  neg_c = jnp.maximum(
      jnp.max(jnp.where(m_negb, s_gn, -_BIG), axis=1, keepdims=True),
      jnp.max(jnp.where(m_negc, s_gg, -_BIG), axis=1, keepdims=True))
  term_c = jnp.where(rep_c, jax.nn.relu(neg_c - pos_c + MARGIN), 0.0)

  n_l = jnp.sum(l_c.astype(f32))
  n_m = jnp.sum(rep_c.astype(f32))
  n_u = jnp.sum(u_c.astype(f32))
  triplet = (jnp.sum(term_b) + jnp.sum(term_c)) / (n_l + n_m)

  tk = jnp.sum(jnp.where(l_c, SCALAR * jnp.sum(nv * g, axis=1, keepdims=True),
                         0.0))
  lmask_s[...] = l_c.astype(f32)
  sio = lax.broadcasted_iota(jnp.int32, (1, 8), 1)
  scal_s[...] = jnp.where(
      sio == 0, triplet,
      jnp.where(sio == 1, tk, jnp.where(sio == 2, n_l, n_u)))

  lowtri = (io_r <= io_c).astype(f32)
  cum_r = lax.dot_general(u_r.astype(f32), lowtri,
                          (((1,), (1,)), ((), ())),
                          preferred_element_type=f32)
  tgt_r = jnp.where(u_r, (BATCH - n_u) + cum_r - 1.0, -5.0)
  onehot_u = (io_c.astype(f32) == tgt_r).astype(f32)
  u_end = lax.dot_general(onehot_u, nv, (((1,), (0,)), ((), ())),
                          preferred_element_type=f32)
  buf_s[0:BATCH, :] = u_end
  buf_s[BATCH:BATCH + CQ_SIZE, :] = cq_ref[...]
  start = BATCH - jnp.sum(u_c.astype(jnp.int32))
  newcq_ref[...] = buf_s[pl.ds(start, CQ_SIZE), :]


def _main_body(vec_ref, labc_ref, labr_ref, g_ref, cq_ref, lut_ref,
               newlut_ref, loss_ref, newcq_ref,
               nv30_s, delta_s, winlab_s, lmask_s, scal_s, se_s, buf_s):
  p = pl.program_id(0)
  f32 = jnp.float32

  @pl.when(p == 0)
  def _():
    _prep_step(vec_ref, labc_ref, labr_ref, g_ref, cq_ref, newcq_ref,
               nv30_s, delta_s, winlab_s, lmask_s, scal_s, se_s, buf_s)

  tile = lut_ref[...]
  s = lax.dot_general(nv30_s[...], tile.astype(jnp.bfloat16),
                      (((1,), (1,)), ((), ())),
                      preferred_element_type=f32)
  se_s[...] += jnp.sum(jnp.exp2(s), axis=1, keepdims=True)

  rows = lax.broadcasted_iota(jnp.int32, (_T, BATCH), 0)
  onehot = (rows == winlab_s[...] - p * _T).astype(jnp.bfloat16)
  scat = lax.dot_general(onehot, delta_s[...], (((1,), (0,)), ((), ())),
                         preferred_element_type=f32)
  newlut_ref[...] = tile + scat

  @pl.when(p == _NLUT - 1)
  def _():
    lse = jnp.log(se_s[...])
    lse_sum = jnp.sum(lmask_s[...] * lse)
    softmax_loss = (lse_sum - scal_s[0, 1]) / scal_s[0, 2]
    loss_ref[...] = jnp.broadcast_to(scal_s[0, 0] + softmax_loss, (1, 1))


def _run_main(vectors, labels, g, cq, lut, interpret=False):
  labc = labels.reshape(BATCH, 1)
  labr = labels.reshape(1, BATCH)
  outs = [
      jax.ShapeDtypeStruct((LUT_SIZE, DIM), jnp.float32),
      jax.ShapeDtypeStruct((1, 1), jnp.float32),
      jax.ShapeDtypeStruct((CQ_SIZE, DIM), jnp.float32),
  ]
  const = lambda p: (0, 0)
  return pl.pallas_call(
      _main_body,
      grid=(_NLUT,),
      in_specs=[
          pl.BlockSpec((BATCH, DIM), const),
          pl.BlockSpec((BATCH, 1), const),
          pl.BlockSpec((1, BATCH), const),
          pl.BlockSpec((BATCH, DIM), const),
          pl.BlockSpec((CQ_SIZE, DIM), const),
          pl.BlockSpec((_T, DIM), lambda p: (p, 0)),
      ],
      out_specs=[
          pl.BlockSpec((_T, DIM), lambda p: (p, 0)),
          pl.BlockSpec((1, 1), const),
          pl.BlockSpec((CQ_SIZE, DIM), const),
      ],
      out_shape=outs,
      scratch_shapes=[
          pltpu.VMEM((BATCH, DIM), jnp.bfloat16),
          pltpu.VMEM((BATCH, DIM), jnp.bfloat16),
          pltpu.VMEM((1, BATCH), jnp.int32),
          pltpu.VMEM((BATCH, 1), jnp.float32),
          pltpu.VMEM((1, 8), jnp.float32),
          pltpu.VMEM((BATCH, 1), jnp.float32),
          pltpu.VMEM((BATCH + CQ_SIZE, DIM), jnp.float32),
      ],
      interpret=interpret,
  )(vectors, labc, labr, g, cq, lut)


def kernel(vectors, labels, lut, cq):
  lab = labels - 1
  l_mask = (labels > 0) & (lab < LUT_SIZE)
  safe_lab = jnp.where(l_mask, lab, 0).astype(jnp.int32)
  g = _sc_gather(lut, safe_lab)
  new_lut, loss, new_cq = _run_main(vectors, labels, g, cq, lut)
  return loss[0, 0], new_lut, new_cq


# speedup vs baseline: 1.0062x; 1.0062x over previous
import functools

import jax
import jax.numpy as jnp
from jax import lax
from jax.experimental import pallas as pl
from jax.experimental.pallas import tpu as pltpu
from jax.experimental.pallas import tpu_sc as plsc

DIM = 64
LUT_SIZE = 100000
CQ_SIZE = 5000
MOMENTUM = 0.5
SCALAR = 30.0
MARGIN = 0.25
BATCH = 1024

_T = 5000
_NLUT = LUT_SIZE // _T
_BIG = 1e30
_LOG2E = 1.4426950408889634


def _sc_gather(lut, safe_lab):
  mesh = plsc.VectorSubcoreMesh(core_axis_name="c", subcore_axis_name="s")
  n_per = BATCH // 32

  @functools.partial(
      pl.kernel,
      out_type=jax.ShapeDtypeStruct((BATCH, DIM), jnp.float32),
      mesh=mesh,
      scratch_types=[
          pltpu.VMEM((n_per,), jnp.int32),
          pltpu.VMEM((n_per, DIM), jnp.float32),
          pltpu.SemaphoreType.DMA,
      ],
  )
  def gather_k(table_hbm, idx_hbm, out_hbm, idx_v, rows_v, sem):
    wid = lax.axis_index("s") * 2 + lax.axis_index("c")
    base = wid * n_per
    pltpu.sync_copy(idx_hbm.at[pl.ds(base, n_per)], idx_v)
    handles = []
    for c in range(n_per // 16):
      vec = idx_v[pl.ds(c * 16, 16)]
      for k in range(16):
        j = c * 16 + k
        handles.append(pltpu.async_copy(
            table_hbm.at[pl.ds(vec[k], 1), :], rows_v.at[pl.ds(j, 1), :],
            sem))
    for h in handles:
      h.wait()
    pltpu.sync_copy(rows_v, out_hbm.at[pl.ds(base, n_per)])

  return gather_k(lut, safe_lab)


def _prep_step(vec_ref, labc_ref, labr_ref, g_ref, cq_ref, newcq_ref,
               nv30_s, delta_s, winlab_s, lmask_s, scal_s, se_s, buf_s):
  f32 = jnp.float32
  v = vec_ref[...]
  nrm = jnp.sqrt(jnp.sum(v * v, axis=1, keepdims=True))
  nv = v / jnp.maximum(nrm, 1e-12)
  g = g_ref[...]
  nv30 = (nv * (SCALAR * _LOG2E)).astype(jnp.bfloat16)
  nv30_s[...] = nv30

  secq = jnp.zeros((BATCH, 1), f32)
  for c in range(CQ_SIZE // 1000):
    chunk = cq_ref[pl.ds(c * 1000, 1000), :].astype(jnp.bfloat16)
    s_c = lax.dot_general(nv30, chunk, (((1,), (1,)), ((), ())),
                          preferred_element_type=f32)
    secq = secq + jnp.sum(jnp.exp2(s_c), axis=1, keepdims=True)
  se_s[...] = secq

  labc = labc_ref[...]
  labr = labr_ref[...]
  lab_c = labc - 1
  lab_r = labr - 1
  l_c = (labc > 0) & (lab_c < LUT_SIZE)
  l_r = (labr > 0) & (lab_r < LUT_SIZE)
  u_c = (labc > 0) & (lab_c >= LUT_SIZE)
  u_r = (labr > 0) & (lab_r >= LUT_SIZE)

  eq = (lab_c == lab_r) & l_c & l_r
  io_c = lax.broadcasted_iota(jnp.int32, (BATCH, BATCH), 0)
  io_r = lax.broadcasted_iota(jnp.int32, (BATCH, BATCH), 1)
  rep_r = l_r & ~jnp.any(eq & (io_c < io_r), axis=0, keepdims=True)
  rep_c = l_c & ~jnp.any(eq & (io_r < io_c), axis=1, keepdims=True)
  win_r = l_r & ~jnp.any(eq & (io_c > io_r), axis=0, keepdims=True)
  winlab_s[...] = jnp.where(win_r, lab_r, -1)

  ratio = (1.0 - MOMENTUM) / MOMENTUM
  w = g + ratio * nv
  wn = jnp.sqrt(jnp.sum(w * w, axis=1, keepdims=True))
  upd = w / jnp.maximum(wn, 1e-12)
  delta_s[...] = (upd - g).astype(jnp.bfloat16)

  dot_t = lambda a, b: lax.dot_general(a, b, (((1,), (1,)), ((), ())),
                                       preferred_element_type=f32)
  s_nn = dot_t(nv, nv)
  s_ng = dot_t(nv, g)
  s_gn = dot_t(g, nv)
  s_gg = dot_t(g, g)

  neq = ~eq
  m_negb = (l_r & neq) | u_r
  m_posc = eq & rep_r
  m_negc = rep_r & neq

  pos_b = jnp.minimum(
      jnp.min(jnp.where(eq, s_nn, _BIG), axis=1, keepdims=True),
      jnp.min(jnp.where(m_posc, s_ng, _BIG), axis=1, keepdims=True))
  neg_b = jnp.maximum(
      jnp.max(jnp.where(m_negb, s_nn, -_BIG), axis=1, keepdims=True),
      jnp.max(jnp.where(m_negc, s_ng, -_BIG), axis=1, keepdims=True))
  term_b = jnp.where(l_c, jax.nn.relu(neg_b - pos_b + MARGIN), 0.0)

  pos_c = jnp.minimum(
      jnp.min(jnp.where(eq, s_gn, _BIG), axis=1, keepdims=True),
      jnp.min(jnp.where(m_posc, s_gg, _BIG), axis=1, keepdims=True))
  neg_c = jnp.maximum(
      jnp.max(jnp.where(m_negb, s_gn, -_BIG), axis=1, keepdims=True),
      jnp.max(jnp.where(m_negc, s_gg, -_BIG), axis=1, keepdims=True))
  term_c = jnp.where(rep_c, jax.nn.relu(neg_c - pos_c + MARGIN), 0.0)

  n_l = jnp.sum(l_c.astype(f32))
  n_m = jnp.sum(rep_c.astype(f32))
  n_u = jnp.sum(u_c.astype(f32))
  triplet = (jnp.sum(term_b) + jnp.sum(term_c)) / (n_l + n_m)

  tk = jnp.sum(jnp.where(l_c, SCALAR * jnp.sum(nv * g, axis=1, keepdims=True),
                         0.0))
  lmask_s[...] = l_c.astype(f32)
  sio = lax.broadcasted_iota(jnp.int32, (1, 8), 1)
  scal_s[...] = jnp.where(
      sio == 0, triplet,
      jnp.where(sio == 1, tk, jnp.where(sio == 2, n_l, n_u)))

  lowtri = (io_r <= io_c).astype(f32)
  cum_r = lax.dot_general(u_r.astype(f32), lowtri,
                          (((1,), (1,)), ((), ())),
                          preferred_element_type=f32)
  tgt_r = jnp.where(u_r, (BATCH - n_u) + cum_r - 1.0, -5.0)
  onehot_u = (io_c.astype(f32) == tgt_r).astype(f32)
  u_end = lax.dot_general(onehot_u, nv, (((1,), (0,)), ((), ())),
                          preferred_element_type=f32)
  buf_s[0:BATCH, :] = u_end
  buf_s[BATCH:BATCH + CQ_SIZE, :] = cq_ref[...]
  start = BATCH - jnp.sum(u_c.astype(jnp.int32))
  newcq_ref[...] = buf_s[pl.ds(start, CQ_SIZE), :]


def _main_body(vec_ref, labc_ref, labr_ref, g_ref, cq_ref, lut_ref,
               newlut_ref, loss_ref, newcq_ref,
               nv30_s, delta_s, winlab_s, lmask_s, scal_s, se_s, buf_s):
  p = pl.program_id(0)
  f32 = jnp.float32

  @pl.when(p == 0)
  def _():
    _prep_step(vec_ref, labc_ref, labr_ref, g_ref, cq_ref, newcq_ref,
               nv30_s, delta_s, winlab_s, lmask_s, scal_s, se_s, buf_s)

  tile = lut_ref[...]
  s = lax.dot_general(nv30_s[...], tile.astype(jnp.bfloat16),
                      (((1,), (1,)), ((), ())),
                      preferred_element_type=f32)
  se_s[...] += jnp.sum(jnp.exp2(s), axis=1, keepdims=True)

  rows = lax.broadcasted_iota(jnp.int32, (_T, BATCH), 0)
  onehot = (rows == winlab_s[...] - p * _T).astype(jnp.bfloat16)
  scat = lax.dot_general(onehot, delta_s[...], (((1,), (0,)), ((), ())),
                         preferred_element_type=f32)
  newlut_ref[...] = tile + scat

  @pl.when(p == _NLUT - 1)
  def _():
    lse = jnp.log(se_s[...])
    lse_sum = jnp.sum(lmask_s[...] * lse)
    softmax_loss = (lse_sum - scal_s[0, 1]) / scal_s[0, 2]
    loss_ref[...] = jnp.broadcast_to(scal_s[0, 0] + softmax_loss, (1, 1))


def _run_main(vectors, labels, g, cq, lut, interpret=False):
  labc = labels.reshape(BATCH, 1)
  labr = labels.reshape(1, BATCH)
  outs = [
      jax.ShapeDtypeStruct((LUT_SIZE, DIM), jnp.float32),
      jax.ShapeDtypeStruct((1, 1), jnp.float32),
      jax.ShapeDtypeStruct((CQ_SIZE, DIM), jnp.float32),
  ]
  const = lambda p: (0, 0)
  return pl.pallas_call(
      _main_body,
      grid=(_NLUT,),
      in_specs=[
          pl.BlockSpec((BATCH, DIM), const),
          pl.BlockSpec((BATCH, 1), const),
          pl.BlockSpec((1, BATCH), const),
          pl.BlockSpec((BATCH, DIM), const),
          pl.BlockSpec((CQ_SIZE, DIM), const),
          pl.BlockSpec((_T, DIM), lambda p: (p, 0)),
      ],
      out_specs=[
          pl.BlockSpec((_T, DIM), lambda p: (p, 0)),
          pl.BlockSpec((1, 1), const),
          pl.BlockSpec((CQ_SIZE, DIM), const),
      ],
      out_shape=outs,
      scratch_shapes=[
          pltpu.VMEM((BATCH, DIM), jnp.bfloat16),
          pltpu.VMEM((BATCH, DIM), jnp.bfloat16),
          pltpu.VMEM((1, BATCH), jnp.int32),
          pltpu.VMEM((BATCH, 1), jnp.float32),
          pltpu.VMEM((1, 8), jnp.float32),
          pltpu.VMEM((BATCH, 1), jnp.float32),
          pltpu.VMEM((BATCH + CQ_SIZE, DIM), jnp.float32),
      ],
      interpret=interpret,
  )(vectors, labc, labr, g, cq, lut)


def kernel(vectors, labels, lut, cq):
  lab = labels - 1
  l_mask = (labels > 0) & (lab < LUT_SIZE)
  safe_lab = jnp.where(l_mask, lab, 0).astype(jnp.int32)
  g = _sc_gather(lut, safe_lab)
  new_lut, loss, new_cq = _run_main(vectors, labels, g, cq, lut)
  return loss[0, 0], new_lut, new_cq


# speedup vs baseline: 1.0075x; 1.0013x over previous
import functools

import jax
import jax.numpy as jnp
from jax import lax
from jax.experimental import pallas as pl
from jax.experimental.pallas import tpu as pltpu
from jax.experimental.pallas import tpu_sc as plsc

DIM = 64
LUT_SIZE = 100000
CQ_SIZE = 5000
MOMENTUM = 0.5
SCALAR = 30.0
MARGIN = 0.25
BATCH = 1024

_T = 5000
_NLUT = LUT_SIZE // _T
_BIG = 1e30
_LOG2E = 1.4426950408889634


def _sc_gather(lut, safe_lab):
  mesh = plsc.VectorSubcoreMesh(core_axis_name="c", subcore_axis_name="s")
  n_per = BATCH // 32

  @functools.partial(
      pl.kernel,
      out_type=jax.ShapeDtypeStruct((BATCH, DIM), jnp.float32),
      mesh=mesh,
      scratch_types=[
          pltpu.VMEM((n_per,), jnp.int32),
          pltpu.VMEM((n_per, DIM), jnp.float32),
          pltpu.SemaphoreType.DMA,
      ],
  )
  def gather_k(table_hbm, idx_hbm, out_hbm, idx_v, rows_v, sem):
    wid = lax.axis_index("s") * 2 + lax.axis_index("c")
    base = wid * n_per
    pltpu.sync_copy(idx_hbm.at[pl.ds(base, n_per)], idx_v)
    handles = []
    for c in range(n_per // 16):
      vec = idx_v[pl.ds(c * 16, 16)]
      for k in range(16):
        j = c * 16 + k
        handles.append(pltpu.async_copy(
            table_hbm.at[pl.ds(vec[k], 1), :], rows_v.at[pl.ds(j, 1), :],
            sem))
    for h in handles:
      h.wait()
    pltpu.sync_copy(rows_v, out_hbm.at[pl.ds(base, n_per)])

  return gather_k(lut, safe_lab)


def _prep_step(vec_ref, labc_ref, labr_ref, g_ref, cq_ref, newcq_ref,
               nv30_s, delta_s, winlab_s, lmask_s, scal_s, se_s, buf_s):
  f32 = jnp.float32
  v = vec_ref[...]
  nrm = jnp.sqrt(jnp.sum(v * v, axis=1, keepdims=True))
  nv = v / jnp.maximum(nrm, 1e-12)
  g = g_ref[...]
  nv30 = (nv * (SCALAR * _LOG2E)).astype(jnp.bfloat16)
  nv30_s[...] = nv30

  secq = jnp.zeros((BATCH, 1), f32)
  for c in range(CQ_SIZE // 1000):
    chunk = cq_ref[pl.ds(c * 1000, 1000), :].astype(jnp.bfloat16)
    s_c = lax.dot_general(nv30, chunk, (((1,), (1,)), ((), ())),
                          preferred_element_type=f32)
    secq = secq + jnp.sum(jnp.exp2(s_c), axis=1, keepdims=True)
  se_s[...] = secq

  labc = labc_ref[...]
  labr = labr_ref[...]
  lab_c = labc - 1
  lab_r = labr - 1
  l_c = (labc > 0) & (lab_c < LUT_SIZE)
  l_r = (labr > 0) & (lab_r < LUT_SIZE)
  u_c = (labc > 0) & (lab_c >= LUT_SIZE)
  u_r = (labr > 0) & (lab_r >= LUT_SIZE)

  eq = (lab_c == lab_r) & l_c & l_r
  io_c = lax.broadcasted_iota(jnp.int32, (BATCH, BATCH), 0)
  io_r = lax.broadcasted_iota(jnp.int32, (BATCH, BATCH), 1)
  rep_r = l_r & ~jnp.any(eq & (io_c < io_r), axis=0, keepdims=True)
  rep_c = l_c & ~jnp.any(eq & (io_r < io_c), axis=1, keepdims=True)
  win_r = l_r & ~jnp.any(eq & (io_c > io_r), axis=0, keepdims=True)
  winlab_s[...] = jnp.where(win_r, lab_r, -1)

  ratio = (1.0 - MOMENTUM) / MOMENTUM
  w = g + ratio * nv
  wn = jnp.sqrt(jnp.sum(w * w, axis=1, keepdims=True))
  upd = w / jnp.maximum(wn, 1e-12)
  delta_s[...] = (upd - g).astype(jnp.bfloat16)

  dot_t = lambda a, b: lax.dot_general(a, b, (((1,), (1,)), ((), ())),
                                       preferred_element_type=f32)
  s_nn = dot_t(nv, nv)
  s_ng = dot_t(nv, g)
  s_gn = dot_t(g, nv)
  s_gg = dot_t(g, g)

  neq = ~eq
  m_negb = (l_r & neq) | u_r
  m_posc = eq & rep_r
  m_negc = rep_r & neq

  pos_b = jnp.minimum(
      jnp.min(jnp.where(eq, s_nn, _BIG), axis=1, keepdims=True),
      jnp.min(jnp.where(m_posc, s_ng, _BIG), axis=1, keepdims=True))
  neg_b = jnp.maximum(
      jnp.max(jnp.where(m_negb, s_nn, -_BIG), axis=1, keepdims=True),
      jnp.max(jnp.where(m_negc, s_ng, -_BIG), axis=1, keepdims=True))
  term_b = jnp.where(l_c, jax.nn.relu(neg_b - pos_b + MARGIN), 0.0)

  pos_c = jnp.minimum(
      jnp.min(jnp.where(eq, s_gn, _BIG), axis=1, keepdims=True),
      jnp.min(jnp.where(m_posc, s_gg, _BIG), axis=1, keepdims=True))
  neg_c = jnp.maximum(
      jnp.max(jnp.where(m_negb, s_gn, -_BIG), axis=1, keepdims=True),
      jnp.max(jnp.where(m_negc, s_gg, -_BIG), axis=1, keepdims=True))
  term_c = jnp.where(rep_c, jax.nn.relu(neg_c - pos_c + MARGIN), 0.0)

  n_l = jnp.sum(l_c.astype(f32))
  n_m = jnp.sum(rep_c.astype(f32))
  n_u = jnp.sum(u_c.astype(f32))
  triplet = (jnp.sum(term_b) + jnp.sum(term_c)) / (n_l + n_m)

  tk = jnp.sum(jnp.where(l_c, SCALAR * jnp.sum(nv * g, axis=1, keepdims=True),
                         0.0))
  lmask_s[...] = l_c.astype(f32)
  sio = lax.broadcasted_iota(jnp.int32, (1, 8), 1)
  scal_s[...] = jnp.where(
      sio == 0, triplet,
      jnp.where(sio == 1, tk, jnp.where(sio == 2, n_l, n_u)))

  lowtri = (io_r <= io_c).astype(f32)
  cum_r = lax.dot_general(u_r.astype(f32), lowtri,
                          (((1,), (1,)), ((), ())),
                          preferred_element_type=f32)
  tgt_r = jnp.where(u_r, (BATCH - n_u) + cum_r - 1.0, -5.0)
  onehot_u = (io_c.astype(f32) == tgt_r).astype(f32)
  u_end = lax.dot_general(onehot_u, nv, (((1,), (0,)), ((), ())),
                          preferred_element_type=f32)
  buf_s[0:BATCH, :] = u_end
  buf_s[BATCH:BATCH + CQ_SIZE, :] = cq_ref[...]
  start = BATCH - jnp.sum(u_c.astype(jnp.int32))
  newcq_ref[...] = buf_s[pl.ds(start, CQ_SIZE), :]


def _main_body(vec_ref, labc_ref, labr_ref, g_ref, cq_ref, lut_ref,
               newlut_ref, loss_ref, newcq_ref,
               nv30_s, delta_s, winlab_s, lmask_s, scal_s, se_s, buf_s):
  p = pl.program_id(0)
  f32 = jnp.float32

  @pl.when(p == 0)
  def _():
    _prep_step(vec_ref, labc_ref, labr_ref, g_ref, cq_ref, newcq_ref,
               nv30_s, delta_s, winlab_s, lmask_s, scal_s, se_s, buf_s)

  tile = lut_ref[...]
  s = lax.dot_general(nv30_s[...], tile.astype(jnp.bfloat16),
                      (((1,), (1,)), ((), ())),
                      preferred_element_type=f32)
  se_s[...] += jnp.sum(jnp.exp2(s), axis=1, keepdims=True)

  rows = lax.broadcasted_iota(jnp.int32, (_T, BATCH), 0)
  onehot = (rows == winlab_s[...] - p * _T).astype(jnp.bfloat16)
  scat = lax.dot_general(onehot, delta_s[...], (((1,), (0,)), ((), ())),
                         preferred_element_type=f32)
  newlut_ref[...] = tile + scat

  @pl.when(p == _NLUT - 1)
  def _():
    lse = jnp.log(se_s[...])
    lse_sum = jnp.sum(lmask_s[...] * lse)
    softmax_loss = (lse_sum - scal_s[0, 1]) / scal_s[0, 2]
    loss_ref[...] = jnp.broadcast_to(scal_s[0, 0] + softmax_loss, (1, 1))


def _run_main(vectors, labels, g, cq, lut):
  labc = labels.reshape(BATCH, 1)
  labr = labels.reshape(1, BATCH)
  outs = [
      jax.ShapeDtypeStruct((LUT_SIZE, DIM), jnp.float32),
      jax.ShapeDtypeStruct((1, 1), jnp.float32),
      jax.ShapeDtypeStruct((CQ_SIZE, DIM), jnp.float32),
  ]
  const = lambda p: (0, 0)
  return pl.pallas_call(
      _main_body,
      grid=(_NLUT,),
      in_specs=[
          pl.BlockSpec((BATCH, DIM), const),
          pl.BlockSpec((BATCH, 1), const),
          pl.BlockSpec((1, BATCH), const),
          pl.BlockSpec((BATCH, DIM), const),
          pl.BlockSpec((CQ_SIZE, DIM), const),
          pl.BlockSpec((_T, DIM), lambda p: (p, 0)),
      ],
      out_specs=[
          pl.BlockSpec((_T, DIM), lambda p: (p, 0)),
          pl.BlockSpec((1, 1), const),
          pl.BlockSpec((CQ_SIZE, DIM), const),
      ],
      out_shape=outs,
      scratch_shapes=[
          pltpu.VMEM((BATCH, DIM), jnp.bfloat16),
          pltpu.VMEM((BATCH, DIM), jnp.bfloat16),
          pltpu.VMEM((1, BATCH), jnp.int32),
          pltpu.VMEM((BATCH, 1), jnp.float32),
          pltpu.VMEM((1, 8), jnp.float32),
          pltpu.VMEM((BATCH, 1), jnp.float32),
          pltpu.VMEM((BATCH + CQ_SIZE, DIM), jnp.float32),
      ],
  )(vectors, labc, labr, g, cq, lut)


def kernel(vectors, labels, lut, cq):
  lab = labels - 1
  l_mask = (labels > 0) & (lab < LUT_SIZE)
  safe_lab = jnp.where(l_mask, lab, 0).astype(jnp.int32)
  g = _sc_gather(lut, safe_lab)
  new_lut, loss, new_cq = _run_main(vectors, labels, g, cq, lut)
  return loss[0, 0], new_lut, new_cq
